# baseline (speedup 1.0000x reference)
import jax
import jax.numpy as jnp
from jax.experimental import pallas as pl
from jax.experimental.pallas import tpu as pltpu

N = 4096
D = 64
KEEP = 21
BLOCK = 512


def _fused_kernel(x_ref, w0_ref, b0_ref, w1_ref, b1_ref, out_ref,
                  h_ref, ht_ref):
    i = pl.program_id(0)

    @pl.when(i == 0)
    def _embed():
        x = x_ref[...]
        h = jnp.dot(x, w0_ref[...].T, preferred_element_type=jnp.float32)
        h = jnp.maximum(h + b0_ref[...], 0.0)
        h = jnp.dot(h, w1_ref[...].T, preferred_element_type=jnp.float32)
        h = h + b1_ref[...]
        nrm = jnp.sqrt(jnp.sum(h * h, axis=1, keepdims=True))
        h = h / jnp.maximum(nrm, 1e-12)
        h_ref[...] = h
        ht_ref[...] = h.T

    hb = h_ref[pl.ds(i * BLOCK, BLOCK), :]
    sim = jnp.dot(hb, ht_ref[...], preferred_element_type=jnp.float32)
    sp = jnp.maximum(sim, 0.0)
    m = jnp.max(sp, axis=1, keepdims=True)
    for _ in range(KEEP - 1):
        m = jnp.max(jnp.where(sp < m, sp, 0.0), axis=1, keepdims=True)
    out_ref[...] = jnp.where(sp >= m, sp, 0.0)


def kernel(x, W0, b0, W1, b1):
    x = x.astype(jnp.float32)
    b0 = b0.reshape(1, D).astype(jnp.float32)
    b1 = b1.reshape(1, D).astype(jnp.float32)
    grid = (N // BLOCK,)
    return pl.pallas_call(
        _fused_kernel,
        grid=grid,
        in_specs=[
            pl.BlockSpec((N, D), lambda i: (0, 0)),
            pl.BlockSpec((D, D), lambda i: (0, 0)),
            pl.BlockSpec((1, D), lambda i: (0, 0)),
            pl.BlockSpec((D, D), lambda i: (0, 0)),
            pl.BlockSpec((1, D), lambda i: (0, 0)),
        ],
        out_specs=pl.BlockSpec((BLOCK, N), lambda i: (i, 0)),
        out_shape=jax.ShapeDtypeStruct((N, N), jnp.float32),
        scratch_shapes=[
            pltpu.VMEM((N, D), jnp.float32),
            pltpu.VMEM((D, N), jnp.float32),
        ],
        compiler_params=pltpu.CompilerParams(
            dimension_semantics=("arbitrary",),
        ),
    )(x, W0.astype(jnp.float32), b0, W1.astype(jnp.float32), b1)


# speedup vs baseline: 1.0155x; 1.0155x over previous
import jax
import jax.numpy as jnp
from jax.experimental import pallas as pl
from jax.experimental.pallas import tpu as pltpu

N = 4096
D = 64
KEEP = 21
BLOCK = 256


def _fused_kernel(x_ref, w0_ref, b0_ref, w1_ref, b1_ref, out_ref,
                  h_ref, ht_ref):
    i = pl.program_id(0)

    @pl.when(i == 0)
    def _embed():
        x = x_ref[...]
        h = jnp.dot(x, w0_ref[...].T, preferred_element_type=jnp.float32)
        h = jnp.maximum(h + b0_ref[...], 0.0)
        h = jnp.dot(h, w1_ref[...].T, preferred_element_type=jnp.float32)
        h = h + b1_ref[...]
        nrm = jnp.sqrt(jnp.sum(h * h, axis=1, keepdims=True))
        h = h / jnp.maximum(nrm, 1e-12)
        h_ref[...] = h
        ht_ref[...] = h.T

    hb = h_ref[pl.ds(i * BLOCK, BLOCK), :]
    sim = jnp.dot(hb, ht_ref[...], preferred_element_type=jnp.float32)
    m = jnp.max(sim, axis=1, keepdims=True)
    for _ in range(KEEP - 1):
        m = jnp.max(jnp.where(sim < m, sim, -2.0), axis=1, keepdims=True)
    m = jnp.maximum(m, 0.0)
    out_ref[...] = jnp.where(sim >= m, sim, 0.0)


def kernel(x, W0, b0, W1, b1):
    x = x.astype(jnp.float32)
    b0 = b0.reshape(1, D).astype(jnp.float32)
    b1 = b1.reshape(1, D).astype(jnp.float32)
    grid = (N // BLOCK,)
    return pl.pallas_call(
        _fused_kernel,
        grid=grid,
        in_specs=[
            pl.BlockSpec((N, D), lambda i: (0, 0)),
            pl.BlockSpec((D, D), lambda i: (0, 0)),
            pl.BlockSpec((1, D), lambda i: (0, 0)),
            pl.BlockSpec((D, D), lambda i: (0, 0)),
            pl.BlockSpec((1, D), lambda i: (0, 0)),
        ],
        out_specs=pl.BlockSpec((BLOCK, N), lambda i: (i, 0)),
        out_shape=jax.ShapeDtypeStruct((N, N), jnp.float32),
        scratch_shapes=[
            pltpu.VMEM((N, D), jnp.float32),
            pltpu.VMEM((D, N), jnp.float32),
        ],
        compiler_params=pltpu.CompilerParams(
            dimension_semantics=("arbitrary",),
        ),
    )(x, W0.astype(jnp.float32), b0, W1.astype(jnp.float32), b1)
